# Initial kernel scaffold; baseline (speedup 1.0000x reference)
import jax
import jax.numpy as jnp
from jax.experimental import pallas as pl


def kernel(x, edge_index_rel0, edge_index_rel1, edge_index_rel2, W0_0, b0_0, W0_1, b0_1, W0_2, b0_2, W1_0, b1_0, W1_1, b1_1, W1_2, b1_2):
    raise NotImplementedError("write your pallas kernel here")


# speedup vs baseline: 4.0106x; 4.0106x over previous
import functools

import jax
import jax.numpy as jnp
from jax import lax
from jax.experimental import pallas as pl
from jax.experimental.pallas import tpu as pltpu
from jax.experimental.pallas import tpu_sc as plsc

N = 10000
D = 128
C = 16
E = 160000
NREL = 3

NCORES = 2
NSUB = 16
NW = NCORES * NSUB
CH = 128
NCH = (E + NW * CH - 1) // (NW * CH)
EPAD = NW * NCH * CH
NPAD = 10240
RPT = NPAD // NSUB

_f32 = jnp.float32


def _sc_mesh():
  return plsc.VectorSubcoreMesh(core_axis_name="c", subcore_axis_name="s")


_SC_PARAMS = pltpu.CompilerParams(use_tc_tiling_on_sc=False)


@functools.partial(
    pl.kernel,
    out_type=jax.ShapeDtypeStruct((NCORES, NPAD, 16), _f32),
    mesh=_sc_mesh(),
    compiler_params=_SC_PARAMS,
    scratch_types=[
        pltpu.VMEM((NCH, CH), jnp.int32),
        pltpu.VMEM((2 * NREL, CH, 16), _f32),
        pltpu.VMEM_SHARED((NPAD, 16), _f32),
    ],
)
def _sc_degrees(src_hbm, dst_hbm, z16_hbm, onehot_hbm, out_hbm,
                idx_v, const_v, acc_sh):
  cid = lax.axis_index("c")
  sid = lax.axis_index("s")
  wid = sid * NCORES + cid
  pltpu.sync_copy(z16_hbm, acc_sh.at[pl.ds(sid * RPT, RPT)])
  pltpu.sync_copy(onehot_hbm, const_v)
  plsc.subcore_barrier()
  for hist_base, e_hbm in ((0, src_hbm), (NREL, dst_hbm)):
    for r in range(NREL):
      pltpu.sync_copy(e_hbm.at[r, wid], idx_v)

      def body(j, carry):
        pltpu.sync_copy(const_v.at[hist_base + r],
                        acc_sh.at[idx_v.at[j]], add=True)
        return carry

      lax.fori_loop(0, NCH, body, 0)
  plsc.subcore_barrier()
  pltpu.sync_copy(acc_sh.at[pl.ds(sid * RPT, RPT)],
                  out_hbm.at[cid, pl.ds(sid * RPT, RPT)])


@functools.partial(
    pl.kernel,
    out_type=jax.ShapeDtypeStruct((NREL, NCORES, NPAD, D), _f32),
    mesh=_sc_mesh(),
    compiler_params=_SC_PARAMS,
    scratch_types=[
        pltpu.VMEM((NCH, CH), jnp.int32),
        pltpu.VMEM((NCH, CH), jnp.int32),
        pltpu.VMEM((CH, D), _f32),
        pltpu.VMEM_SHARED((NPAD, D), _f32),
    ],
)
def _sc_layer0(f0, f1, f2, src_hbm, dst_hbm, z128_hbm, out_hbm,
               idx_s, idx_d, rows_v, acc_sh):
  cid = lax.axis_index("c")
  sid = lax.axis_index("s")
  wid = sid * NCORES + cid
  feats = (f0, f1, f2)
  for r in range(NREL):
    pltpu.sync_copy(z128_hbm, acc_sh.at[pl.ds(sid * RPT, RPT)])
    pltpu.sync_copy(src_hbm.at[r, wid], idx_s)
    pltpu.sync_copy(dst_hbm.at[r, wid], idx_d)
    plsc.subcore_barrier()

    def body(j, carry, _r=r):
      pltpu.sync_copy(feats[_r].at[idx_s.at[j]], rows_v)
      pltpu.sync_copy(rows_v, acc_sh.at[idx_d.at[j]], add=True)
      return carry

    lax.fori_loop(0, NCH, body, 0)
    plsc.subcore_barrier()
    pltpu.sync_copy(acc_sh.at[pl.ds(sid * RPT, RPT)],
                    out_hbm.at[r, cid, pl.ds(sid * RPT, RPT)])


@functools.partial(
    pl.kernel,
    out_type=jax.ShapeDtypeStruct((NREL, NCORES, NPAD, C), _f32),
    mesh=_sc_mesh(),
    compiler_params=_SC_PARAMS,
    scratch_types=[
        pltpu.VMEM((NCH, CH), jnp.int32),
        pltpu.VMEM((NCH, CH), jnp.int32),
        pltpu.VMEM((CH, C), _f32),
        pltpu.VMEM_SHARED((NPAD, C), _f32),
        pltpu.VMEM_SHARED((NPAD, C), _f32),
        pltpu.VMEM_SHARED((NPAD, C), _f32),
    ],
)
def _sc_layer1(y0, y1, y2, src_hbm, dst_hbm, z16_hbm, out_hbm,
               idx_s, idx_d, rows_v, a0, a1, a2):
  cid = lax.axis_index("c")
  sid = lax.axis_index("s")
  wid = sid * NCORES + cid
  ys = (y0, y1, y2)
  accs = (a0, a1, a2)
  for r in range(NREL):
    pltpu.sync_copy(z16_hbm, accs[r].at[pl.ds(sid * RPT, RPT)])
  plsc.subcore_barrier()
  for r in range(NREL):
    pltpu.sync_copy(src_hbm.at[r, wid], idx_s)
    pltpu.sync_copy(dst_hbm.at[r, wid], idx_d)

    def body(j, carry, _r=r):
      pltpu.sync_copy(ys[_r].at[idx_s.at[j]], rows_v)
      pltpu.sync_copy(rows_v, accs[_r].at[idx_d.at[j]], add=True)
      return carry

    lax.fori_loop(0, NCH, body, 0)
  plsc.subcore_barrier()
  for r in range(NREL):
    pltpu.sync_copy(accs[r].at[pl.ds(sid * RPT, RPT)],
                    out_hbm.at[r, cid, pl.ds(sid * RPT, RPT)])


_RB = 1024


def _scale_body(x_ref, degp_ref, o0, o1, o2):
  x = x_ref[...]
  d = degp_ref[0] + degp_ref[1]
  for r, o in enumerate((o0, o1, o2)):
    s = lax.rsqrt(jnp.maximum(d[:, r], 1.0))
    o[...] = x * s[:, None]


def _tc_scale(x_pad, degp):
  grid = NPAD // _RB
  outs = [jax.ShapeDtypeStruct((NPAD, D), _f32)] * NREL
  return pl.pallas_call(
      _scale_body,
      grid=(grid,),
      in_specs=[
          pl.BlockSpec((_RB, D), lambda i: (i, 0)),
          pl.BlockSpec((NCORES, _RB, 16), lambda i: (0, i, 0)),
      ],
      out_specs=[pl.BlockSpec((_RB, D), lambda i: (i, 0))] * NREL,
      out_shape=outs,
  )(x_pad, degp)


def _mid_body(agg_ref, degp_ref, w0_ref, b0_ref, w1_ref, y0, y1, y2):
  d = degp_ref[0] + degp_ref[1]
  h = jnp.zeros((_RB, D), _f32)
  for r in range(NREL):
    din = lax.rsqrt(jnp.maximum(d[:, NREL + r], 1.0))
    a = (agg_ref[r, 0] + agg_ref[r, 1]) * din[:, None]
    h = h + jnp.dot(a, w0_ref[r], preferred_element_type=_f32)
  h = h + (b0_ref[0] + b0_ref[1] + b0_ref[2])[None, :]
  h = jnp.maximum(h, 0.0)
  for r, y in enumerate((y0, y1, y2)):
    dout = lax.rsqrt(jnp.maximum(d[:, r], 1.0))
    y[...] = jnp.dot(h * dout[:, None], w1_ref[r],
                     preferred_element_type=_f32)


def _tc_mid(agg, degp, w0s, b0s, w1s):
  grid = NPAD // _RB
  outs = [jax.ShapeDtypeStruct((NPAD, C), _f32)] * NREL
  return pl.pallas_call(
      _mid_body,
      grid=(grid,),
      in_specs=[
          pl.BlockSpec((NREL, NCORES, _RB, D), lambda i: (0, 0, i, 0)),
          pl.BlockSpec((NCORES, _RB, 16), lambda i: (0, i, 0)),
          pl.BlockSpec((NREL, D, D), lambda i: (0, 0, 0)),
          pl.BlockSpec((NREL, D), lambda i: (0, 0)),
          pl.BlockSpec((NREL, D, C), lambda i: (0, 0, 0)),
      ],
      out_specs=[pl.BlockSpec((_RB, C), lambda i: (i, 0))] * NREL,
      out_shape=outs,
  )(agg, degp, w0s, b0s, w1s)


def _fin_body(yp_ref, degp_ref, b1_ref, out_ref):
  d = degp_ref[0] + degp_ref[1]
  acc = jnp.broadcast_to((b1_ref[0] + b1_ref[1] + b1_ref[2])[None, :],
                         (_RB, C))
  for r in range(NREL):
    din = lax.rsqrt(jnp.maximum(d[:, NREL + r], 1.0))
    acc = acc + (yp_ref[r, 0] + yp_ref[r, 1]) * din[:, None]
  out_ref[...] = acc


def _tc_final(yp, degp, b1s):
  grid = NPAD // _RB
  return pl.pallas_call(
      _fin_body,
      grid=(grid,),
      in_specs=[
          pl.BlockSpec((NREL, NCORES, _RB, C), lambda i: (0, 0, i, 0)),
          pl.BlockSpec((NCORES, _RB, 16), lambda i: (0, i, 0)),
          pl.BlockSpec((NREL, C), lambda i: (0, 0)),
      ],
      out_specs=pl.BlockSpec((_RB, C), lambda i: (i, 0)),
      out_shape=jax.ShapeDtypeStruct((NPAD, C), _f32),
  )(yp, degp, b1s)


def _prep_idx(ei):
  pad = EPAD - E
  out = []
  for k in range(2):
    v = ei[k].astype(jnp.int32)
    v = jnp.concatenate([v, jnp.full((pad,), N, jnp.int32)])
    out.append(v.reshape(NW, NCH, CH))
  return out[0], out[1]


def kernel(x, edge_index_rel0, edge_index_rel1, edge_index_rel2,
           W0_0, b0_0, W0_1, b0_1, W0_2, b0_2,
           W1_0, b1_0, W1_1, b1_1, W1_2, b1_2):
  srcs, dsts = [], []
  for ei in (edge_index_rel0, edge_index_rel1, edge_index_rel2):
    s, t = _prep_idx(ei)
    srcs.append(s)
    dsts.append(t)
  src_all = jnp.stack(srcs)
  dst_all = jnp.stack(dsts)

  x_pad = jnp.zeros((NPAD, D), _f32).at[:N].set(x)

  z16 = jnp.zeros((RPT, 16), _f32)
  z128 = jnp.zeros((RPT, D), _f32)
  onehot = jnp.broadcast_to(
      jnp.eye(16, dtype=_f32)[:2 * NREL, None, :], (2 * NREL, CH, 16)
  ).copy()

  w0s = jnp.stack([W0_0, W0_1, W0_2])
  b0s = jnp.stack([b0_0, b0_1, b0_2])
  w1s = jnp.stack([W1_0, W1_1, W1_2])
  b1s = jnp.stack([b1_0, b1_1, b1_2])

  degp = _sc_degrees(src_all, dst_all, z16, onehot)
  f0, f1, f2 = _tc_scale(x_pad, degp)
  agg = _sc_layer0(f0, f1, f2, src_all, dst_all, z128)
  y0, y1, y2 = _tc_mid(agg, degp, w0s, b0s, w1s)
  yp = _sc_layer1(y0, y1, y2, src_all, dst_all, z16)
  out_pad = _tc_final(yp, degp, b1s)
  return out_pad[:N]


# speedup vs baseline: 4.5753x; 1.1408x over previous
import functools

import jax
import jax.numpy as jnp
from jax import lax
from jax.experimental import pallas as pl
from jax.experimental.pallas import tpu as pltpu
from jax.experimental.pallas import tpu_sc as plsc

N = 10000
D = 128
C = 16
E = 160000
NREL = 3

NCORES = 2
NSUB = 16
NW = NCORES * NSUB
CH = 128
NCH = (E + NW * CH - 1) // (NW * CH)
EPAD = NW * NCH * CH
NPAD = 10240
RPT = NPAD // NSUB

_f32 = jnp.float32


def _sc_mesh():
  return plsc.VectorSubcoreMesh(core_axis_name="c", subcore_axis_name="s")


_SC_PARAMS = pltpu.CompilerParams(use_tc_tiling_on_sc=False)


@functools.partial(
    pl.kernel,
    out_type=jax.ShapeDtypeStruct((NCORES, NPAD, 16), _f32),
    mesh=_sc_mesh(),
    compiler_params=_SC_PARAMS,
    scratch_types=[
        pltpu.VMEM((NCH, CH), jnp.int32),
        pltpu.VMEM((2 * NREL, CH, 16), _f32),
        pltpu.VMEM_SHARED((NPAD, 16), _f32),
    ],
)
def _sc_degrees(src_hbm, dst_hbm, z16_hbm, onehot_hbm, out_hbm,
                idx_v, const_v, acc_sh):
  cid = lax.axis_index("c")
  sid = lax.axis_index("s")
  wid = sid * NCORES + cid
  pltpu.sync_copy(z16_hbm, acc_sh.at[pl.ds(sid * RPT, RPT)])
  pltpu.sync_copy(onehot_hbm, const_v)
  plsc.subcore_barrier()
  for hist_base, e_hbm in ((0, src_hbm), (NREL, dst_hbm)):
    for r in range(NREL):
      pltpu.sync_copy(e_hbm.at[r, wid], idx_v)

      def body(j, carry):
        pltpu.sync_copy(const_v.at[hist_base + r],
                        acc_sh.at[idx_v.at[j]], add=True)
        return carry

      lax.fori_loop(0, NCH, body, 0)
  plsc.subcore_barrier()
  pltpu.sync_copy(acc_sh.at[pl.ds(sid * RPT, RPT)],
                  out_hbm.at[cid, pl.ds(sid * RPT, RPT)])


def _edge_pass(table, idx_s, idx_d, rows0, rows1, gsem0, gsem1, acc):
  pltpu.async_copy(table.at[idx_s.at[0]], rows0, gsem0)

  def body(t, carry):
    j = 2 * t
    pltpu.async_copy(table.at[idx_s.at[j + 1]], rows1, gsem1)
    pltpu.make_async_copy(table.at[idx_s.at[j]], rows0, gsem0).wait()
    pltpu.sync_copy(rows0, acc.at[idx_d.at[j]], add=True)
    jn = jnp.minimum(j + 2, NCH - 1)
    pltpu.async_copy(table.at[idx_s.at[jn]], rows0, gsem0)
    pltpu.make_async_copy(table.at[idx_s.at[j + 1]], rows1, gsem1).wait()
    pltpu.sync_copy(rows1, acc.at[idx_d.at[j + 1]], add=True)
    return carry

  lax.fori_loop(0, NCH // 2, body, 0)
  pltpu.make_async_copy(table.at[idx_s.at[NCH - 1]], rows0, gsem0).wait()


@functools.partial(
    pl.kernel,
    out_type=jax.ShapeDtypeStruct((NREL, NCORES, NPAD, D), _f32),
    mesh=_sc_mesh(),
    compiler_params=_SC_PARAMS,
    scratch_types=[
        pltpu.VMEM((NCH, CH), jnp.int32),
        pltpu.VMEM((NCH, CH), jnp.int32),
        pltpu.VMEM((CH, D), _f32),
        pltpu.VMEM((CH, D), _f32),
        pltpu.SemaphoreType.DMA,
        pltpu.SemaphoreType.DMA,
        pltpu.VMEM_SHARED((NPAD, D), _f32),
    ],
)
def _sc_layer0(f0, f1, f2, src_hbm, dst_hbm, z128_hbm, out_hbm,
               idx_s, idx_d, rows0, rows1, gsem0, gsem1, acc_sh):
  cid = lax.axis_index("c")
  sid = lax.axis_index("s")
  wid = sid * NCORES + cid
  feats = (f0, f1, f2)
  for r in range(NREL):
    pltpu.sync_copy(z128_hbm, acc_sh.at[pl.ds(sid * RPT, RPT)])
    pltpu.sync_copy(src_hbm.at[r, wid], idx_s)
    pltpu.sync_copy(dst_hbm.at[r, wid], idx_d)
    plsc.subcore_barrier()
    _edge_pass(feats[r], idx_s, idx_d, rows0, rows1, gsem0, gsem1, acc_sh)
    plsc.subcore_barrier()
    pltpu.sync_copy(acc_sh.at[pl.ds(sid * RPT, RPT)],
                    out_hbm.at[r, cid, pl.ds(sid * RPT, RPT)])


@functools.partial(
    pl.kernel,
    out_type=jax.ShapeDtypeStruct((NREL, NCORES, NPAD, C), _f32),
    mesh=_sc_mesh(),
    compiler_params=_SC_PARAMS,
    scratch_types=[
        pltpu.VMEM((NCH, CH), jnp.int32),
        pltpu.VMEM((NCH, CH), jnp.int32),
        pltpu.VMEM((CH, C), _f32),
        pltpu.VMEM((CH, C), _f32),
        pltpu.SemaphoreType.DMA,
        pltpu.SemaphoreType.DMA,
        pltpu.VMEM_SHARED((NPAD, C), _f32),
        pltpu.VMEM_SHARED((NPAD, C), _f32),
        pltpu.VMEM_SHARED((NPAD, C), _f32),
    ],
)
def _sc_layer1(y0, y1, y2, src_hbm, dst_hbm, z16_hbm, out_hbm,
               idx_s, idx_d, rows0, rows1, gsem0, gsem1, a0, a1, a2):
  cid = lax.axis_index("c")
  sid = lax.axis_index("s")
  wid = sid * NCORES + cid
  ys = (y0, y1, y2)
  accs = (a0, a1, a2)
  for r in range(NREL):
    pltpu.sync_copy(z16_hbm, accs[r].at[pl.ds(sid * RPT, RPT)])
  plsc.subcore_barrier()
  for r in range(NREL):
    pltpu.sync_copy(src_hbm.at[r, wid], idx_s)
    pltpu.sync_copy(dst_hbm.at[r, wid], idx_d)
    _edge_pass(ys[r], idx_s, idx_d, rows0, rows1, gsem0, gsem1, accs[r])
  plsc.subcore_barrier()
  for r in range(NREL):
    pltpu.sync_copy(accs[r].at[pl.ds(sid * RPT, RPT)],
                    out_hbm.at[r, cid, pl.ds(sid * RPT, RPT)])


_RB = 1024


def _scale_body(x_ref, degp_ref, o0, o1, o2):
  x = x_ref[...]
  d = degp_ref[0] + degp_ref[1]
  for r, o in enumerate((o0, o1, o2)):
    s = lax.rsqrt(jnp.maximum(d[:, r], 1.0))
    o[...] = x * s[:, None]


def _tc_scale(x_pad, degp):
  grid = NPAD // _RB
  outs = [jax.ShapeDtypeStruct((NPAD, D), _f32)] * NREL
  return pl.pallas_call(
      _scale_body,
      grid=(grid,),
      in_specs=[
          pl.BlockSpec((_RB, D), lambda i: (i, 0)),
          pl.BlockSpec((NCORES, _RB, 16), lambda i: (0, i, 0)),
      ],
      out_specs=[pl.BlockSpec((_RB, D), lambda i: (i, 0))] * NREL,
      out_shape=outs,
  )(x_pad, degp)


def _mid_body(agg_ref, degp_ref, w0_ref, b0_ref, w1_ref, y0, y1, y2):
  d = degp_ref[0] + degp_ref[1]
  h = jnp.zeros((_RB, D), _f32)
  for r in range(NREL):
    din = lax.rsqrt(jnp.maximum(d[:, NREL + r], 1.0))
    a = (agg_ref[r, 0] + agg_ref[r, 1]) * din[:, None]
    h = h + jnp.dot(a, w0_ref[r], preferred_element_type=_f32)
  h = h + (b0_ref[0] + b0_ref[1] + b0_ref[2])[None, :]
  h = jnp.maximum(h, 0.0)
  for r, y in enumerate((y0, y1, y2)):
    dout = lax.rsqrt(jnp.maximum(d[:, r], 1.0))
    y[...] = jnp.dot(h * dout[:, None], w1_ref[r],
                     preferred_element_type=_f32)


def _tc_mid(agg, degp, w0s, b0s, w1s):
  grid = NPAD // _RB
  outs = [jax.ShapeDtypeStruct((NPAD, C), _f32)] * NREL
  return pl.pallas_call(
      _mid_body,
      grid=(grid,),
      in_specs=[
          pl.BlockSpec((NREL, NCORES, _RB, D), lambda i: (0, 0, i, 0)),
          pl.BlockSpec((NCORES, _RB, 16), lambda i: (0, i, 0)),
          pl.BlockSpec((NREL, D, D), lambda i: (0, 0, 0)),
          pl.BlockSpec((NREL, D), lambda i: (0, 0)),
          pl.BlockSpec((NREL, D, C), lambda i: (0, 0, 0)),
      ],
      out_specs=[pl.BlockSpec((_RB, C), lambda i: (i, 0))] * NREL,
      out_shape=outs,
  )(agg, degp, w0s, b0s, w1s)


def _fin_body(yp_ref, degp_ref, b1_ref, out_ref):
  d = degp_ref[0] + degp_ref[1]
  acc = jnp.broadcast_to((b1_ref[0] + b1_ref[1] + b1_ref[2])[None, :],
                         (_RB, C))
  for r in range(NREL):
    din = lax.rsqrt(jnp.maximum(d[:, NREL + r], 1.0))
    acc = acc + (yp_ref[r, 0] + yp_ref[r, 1]) * din[:, None]
  out_ref[...] = acc


def _tc_final(yp, degp, b1s):
  grid = NPAD // _RB
  return pl.pallas_call(
      _fin_body,
      grid=(grid,),
      in_specs=[
          pl.BlockSpec((NREL, NCORES, _RB, C), lambda i: (0, 0, i, 0)),
          pl.BlockSpec((NCORES, _RB, 16), lambda i: (0, i, 0)),
          pl.BlockSpec((NREL, C), lambda i: (0, 0)),
      ],
      out_specs=pl.BlockSpec((_RB, C), lambda i: (i, 0)),
      out_shape=jax.ShapeDtypeStruct((NPAD, C), _f32),
  )(yp, degp, b1s)


def _prep_idx(ei):
  pad = EPAD - E
  out = []
  for k in range(2):
    v = ei[k].astype(jnp.int32)
    v = jnp.concatenate([v, jnp.full((pad,), N, jnp.int32)])
    out.append(v.reshape(NW, NCH, CH))
  return out[0], out[1]


def kernel(x, edge_index_rel0, edge_index_rel1, edge_index_rel2,
           W0_0, b0_0, W0_1, b0_1, W0_2, b0_2,
           W1_0, b1_0, W1_1, b1_1, W1_2, b1_2):
  srcs, dsts = [], []
  for ei in (edge_index_rel0, edge_index_rel1, edge_index_rel2):
    s, t = _prep_idx(ei)
    srcs.append(s)
    dsts.append(t)
  src_all = jnp.stack(srcs)
  dst_all = jnp.stack(dsts)

  x_pad = jnp.zeros((NPAD, D), _f32).at[:N].set(x)

  z16 = jnp.zeros((RPT, 16), _f32)
  z128 = jnp.zeros((RPT, D), _f32)
  onehot = jnp.broadcast_to(
      jnp.eye(16, dtype=_f32)[:2 * NREL, None, :], (2 * NREL, CH, 16)
  ).copy()

  w0s = jnp.stack([W0_0, W0_1, W0_2])
  b0s = jnp.stack([b0_0, b0_1, b0_2])
  w1s = jnp.stack([W1_0, W1_1, W1_2])
  b1s = jnp.stack([b1_0, b1_1, b1_2])

  degp = _sc_degrees(src_all, dst_all, z16, onehot)
  f0, f1, f2 = _tc_scale(x_pad, degp)
  agg = _sc_layer0(f0, f1, f2, src_all, dst_all, z128)
  y0, y1, y2 = _tc_mid(agg, degp, w0s, b0s, w1s)
  yp = _sc_layer1(y0, y1, y2, src_all, dst_all, z16)
  out_pad = _tc_final(yp, degp, b1s)
  return out_pad[:N]
